# Initial kernel scaffold; baseline (speedup 1.0000x reference)
import jax
import jax.numpy as jnp
from jax.experimental import pallas as pl


def kernel(x, edge_index, batch, epoch, W_pre, b_pre, W_nbr1, W_self1, b1, W_score_nbr, W_score_self, b_score, W_nbr2, W_self2, b2, W_post, b_post):
    raise NotImplementedError("write your pallas kernel here")


# speedup vs baseline: 1.4795x; 1.4795x over previous
import functools

import jax
import jax.numpy as jnp
from jax import lax
from jax.experimental import pallas as pl
from jax.experimental.pallas import tpu as pltpu
from jax.experimental.pallas import tpu_sc as plsc

N = 10000
E = 320000
K = 5000
H = 256
EP = 327680
NCHUNK = 2560
NACC = 10240
DUMMY = N
F32 = jnp.float32
I32 = jnp.int32


def _tc3_body(score_col_ref, score_row_ref, h1_ref, h0_ref, wn2_ref, ws2_ref,
              b2_ref, xp_idx_ref, nmap_ref, xu1_ref, xu0_ref, p2a_ref, p2b_ref,
              s2_ref):
    i = pl.program_id(0)
    si = score_col_ref[...]
    iglob = lax.broadcasted_iota(I32, (1000, 1), 0) + i * 1000

    def jstep(c, rank):
        sj = score_row_ref[:, pl.ds(c * 1280, 1280)]
        jglob = lax.broadcasted_iota(I32, (1, 1280), 1) + c * 1280
        m = (sj > si) | ((sj == si) & (jglob < iglob))
        return rank + jnp.sum(m.astype(I32), axis=1, keepdims=True)

    rank = lax.fori_loop(0, 8, jstep, jnp.zeros((1000, 1), I32))
    valid = rank < K
    g = jnp.where(valid, jnp.tanh(si), 0.0)
    xp_idx_ref[...] = jnp.where(valid, rank, K + (rank & 7))
    nmap_ref[...] = jnp.where(valid, rank, -1)
    xu1 = h1_ref[...] * g
    xu0 = h0_ref[...] * g
    xu1_ref[...] = xu1
    xu0_ref[...] = xu0
    p2 = (jnp.dot(xu1, wn2_ref[:H], preferred_element_type=F32)
          + jnp.dot(xu0, wn2_ref[H:], preferred_element_type=F32))
    p2a_ref[...] = p2[:, :128]
    p2b_ref[...] = p2[:, 128:]
    s2_ref[...] = (jnp.dot(xu1, ws2_ref[:H], preferred_element_type=F32)
                   + jnp.dot(xu0, ws2_ref[H:], preferred_element_type=F32)
                   + b2_ref[...])


def _tc3(score, h1, h0, W_nbr2, W_self2, b2):
    score_col = score.reshape(N, 1)
    score_row = jnp.concatenate(
        [score, jnp.full((240,), -1e30, F32)]).reshape(1, N + 240)
    return pl.pallas_call(
        _tc3_body,
        grid=(10,),
        in_specs=[
            pl.BlockSpec((1000, 1), lambda i: (i, 0)),
            pl.BlockSpec((1, N + 240), lambda i: (0, 0)),
            pl.BlockSpec((1000, H), lambda i: (i, 0)),
            pl.BlockSpec((1000, H), lambda i: (i, 0)),
            pl.BlockSpec((2 * H, H), lambda i: (0, 0)),
            pl.BlockSpec((2 * H, H), lambda i: (0, 0)),
            pl.BlockSpec((1, H), lambda i: (0, 0)),
        ],
        out_specs=[
            pl.BlockSpec((1000, 1), lambda i: (i, 0)),
            pl.BlockSpec((1000, 1), lambda i: (i, 0)),
            pl.BlockSpec((1000, H), lambda i: (i, 0)),
            pl.BlockSpec((1000, H), lambda i: (i, 0)),
            pl.BlockSpec((1000, 128), lambda i: (i, 0)),
            pl.BlockSpec((1000, 128), lambda i: (i, 0)),
            pl.BlockSpec((1000, H), lambda i: (i, 0)),
        ],
        out_shape=[
            jax.ShapeDtypeStruct((N, 1), I32),
            jax.ShapeDtypeStruct((N, 1), I32),
            jax.ShapeDtypeStruct((N, H), F32),
            jax.ShapeDtypeStruct((N, H), F32),
            jax.ShapeDtypeStruct((N, 128), F32),
            jax.ShapeDtypeStruct((N, 128), F32),
            jax.ShapeDtypeStruct((N, H), F32),
        ],
    )(score_col, score_row, h1, h0, W_nbr2, W_self2, b2.reshape(1, H))


_MESH = plsc.VectorSubcoreMesh(core_axis_name="c", subcore_axis_name="s",
                               num_cores=2, num_subcores=16)


def _segsum_work(tab, src2d, dst2d, zeros, out, acc, srcb, dstb, rowb, s):
    pltpu.sync_copy(zeros, acc.at[pl.ds(s * 640, 640)])
    plsc.subcore_barrier()

    def group(gi, carry):
        base = s * 160 + gi * 16
        pltpu.sync_copy(src2d.at[pl.ds(base, 16)], srcb)
        pltpu.sync_copy(dst2d.at[pl.ds(base, 16)], dstb)

        def body(j, c2):
            pltpu.sync_copy(tab.at[srcb.at[j]], rowb)
            pltpu.sync_copy(rowb, acc.at[dstb.at[j]], add=True)
            return c2

        lax.fori_loop(0, 16, body, 0)
        return carry

    lax.fori_loop(0, 10, group, 0)
    plsc.subcore_barrier()
    pltpu.sync_copy(acc.at[pl.ds(s * 640, 640)], out.at[pl.ds(s * 640, 640)])


@functools.partial(
    pl.kernel,
    out_type=(jax.ShapeDtypeStruct((NACC, 128), F32),
              jax.ShapeDtypeStruct((NACC, 128), F32)),
    mesh=_MESH,
    scratch_types=[
        pltpu.VMEM_SHARED((NACC, 128), F32),
        pltpu.VMEM((16, 128), I32),
        pltpu.VMEM((16, 128), I32),
        pltpu.VMEM((128, 128), F32),
    ],
)
def _sc_segsum(pa, pb, src2d, dst2d, zeros, out_a, out_b, acc, srcb, dstb,
               rowb):
    c = lax.axis_index("c")
    s = lax.axis_index("s")

    @pl.when(c == 0)
    def _():
        _segsum_work(pa, src2d, dst2d, zeros, out_a, acc, srcb, dstb, rowb, s)

    @pl.when(c == 1)
    def _():
        _segsum_work(pb, src2d, dst2d, zeros, out_b, acc, srcb, dstb, rowb, s)


@functools.partial(
    pl.kernel,
    out_type=(jax.ShapeDtypeStruct((K + 8,), I32),
              jax.ShapeDtypeStruct((K + 8, H), F32),
              jax.ShapeDtypeStruct((K + 8, H), F32)),
    mesh=_MESH,
    scratch_types=[
        pltpu.VMEM((4, 80), I32),
        pltpu.VMEM((80,), I32),
        pltpu.VMEM((80, H), F32),
    ],
)
def _sc_pool(xp_idx, ivals, xu1, xu0, s_out, xpa, xpb, idxb, ivb, rowb):
    c = lax.axis_index("c")
    s = lax.axis_index("s")
    wid = s * 2 + c

    def chunk(j, carry):
        cid = wid + 32 * j

        @pl.when(cid < 125)
        def _():
            base = cid * 80
            pltpu.sync_copy(xp_idx.at[pl.ds(base, 80)], idxb.at[j])
            pltpu.sync_copy(ivals.at[pl.ds(base, 80)], ivb)
            pltpu.sync_copy(ivb, s_out.at[idxb.at[j]])
            pltpu.sync_copy(xu1.at[pl.ds(base, 80)], rowb)
            pltpu.sync_copy(rowb, xpa.at[idxb.at[j]])
            pltpu.sync_copy(xu0.at[pl.ds(base, 80)], rowb)
            pltpu.sync_copy(rowb, xpb.at[idxb.at[j]])

        return carry

    lax.fori_loop(0, 4, chunk, 0)


@functools.partial(
    pl.kernel,
    out_type=jax.ShapeDtypeStruct((2, EP), I32),
    mesh=_MESH,
    scratch_types=[
        pltpu.VMEM_SHARED((NACC,), I32),
        pltpu.VMEM((16, 128), I32),
        pltpu.VMEM((16, 128), I32),
        pltpu.VMEM((128,), I32),
        pltpu.VMEM((128,), I32),
        pltpu.VMEM((10240,), I32),
        pltpu.VMEM((10240,), I32),
    ],
)
def _sc_apool(nm_pad, src2d, dst2d, a_out, nmtab, sbuf, dbuf, smb, dmb, osb,
              odb):
    c = lax.axis_index("c")
    s = lax.axis_index("s")
    wid = s * 2 + c

    @pl.when(s == 0)
    def _():
        pltpu.sync_copy(nm_pad, nmtab)

    plsc.subcore_barrier()
    base_cid = wid * 80

    def group(g, carry):
        pltpu.sync_copy(src2d.at[pl.ds(base_cid + g * 16, 16)], sbuf)
        pltpu.sync_copy(dst2d.at[pl.ds(base_cid + g * 16, 16)], dbuf)

        def chunk(j, c2):
            pltpu.sync_copy(nmtab.at[sbuf.at[j]], smb)
            pltpu.sync_copy(nmtab.at[dbuf.at[j]], dmb)

            def vec(t, c3):
                o = t * 16
                sm = smb[pl.ds(o, 16)]
                dm = dmb[pl.ds(o, 16)]
                ok = (sm >= 0) & (dm >= 0)
                off = (g * 16 + j) * 128 + o
                osb[pl.ds(off, 16)] = jnp.where(ok, sm, -1)
                odb[pl.ds(off, 16)] = jnp.where(ok, dm, -1)
                return c3

            lax.fori_loop(0, 8, vec, 0)
            return c2

        lax.fori_loop(0, 16, chunk, 0)
        return carry

    lax.fori_loop(0, 5, group, 0)
    pltpu.sync_copy(osb, a_out.at[0, pl.ds(wid * 10240, 10240)])
    pltpu.sync_copy(odb, a_out.at[1, pl.ds(wid * 10240, 10240)])


def _tc4_body(agga_ref, aggb_ref, s2_ref, xu1_ref, xu0_ref, wp_ref, bp_ref,
              out_ref):
    agg = jnp.concatenate([agga_ref[...], aggb_ref[...]], axis=1)
    h2 = jax.nn.relu(agg + s2_ref[...])
    out_ref[...] = (jnp.dot(h2, wp_ref[:H], preferred_element_type=F32)
                    + jnp.dot(xu1_ref[...], wp_ref[H:2 * H],
                              preferred_element_type=F32)
                    + jnp.dot(xu0_ref[...], wp_ref[2 * H:],
                              preferred_element_type=F32)
                    + bp_ref[...])


def _tc4(agga, aggb, s2, xu1, xu0, W_post, b_post):
    return pl.pallas_call(
        _tc4_body,
        grid=(10,),
        in_specs=[
            pl.BlockSpec((1000, 128), lambda i: (i, 0)),
            pl.BlockSpec((1000, 128), lambda i: (i, 0)),
            pl.BlockSpec((1000, H), lambda i: (i, 0)),
            pl.BlockSpec((1000, H), lambda i: (i, 0)),
            pl.BlockSpec((1000, H), lambda i: (i, 0)),
            pl.BlockSpec((3 * H, 128), lambda i: (0, 0)),
            pl.BlockSpec((1, 128), lambda i: (0, 0)),
        ],
        out_specs=pl.BlockSpec((1000, 128), lambda i: (i, 0)),
        out_shape=jax.ShapeDtypeStruct((N, 128), F32),
    )(agga, aggb, s2, xu1, xu0, W_post, b_post.reshape(1, 128))


def kernel(x, edge_index, batch, epoch, W_pre, b_pre, W_nbr1, W_self1, b1,
           W_score_nbr, W_score_self, b_score, W_nbr2, W_self2, b2, W_post,
           b_post):
    src, dst = edge_index[0], edge_index[1]

    h0 = jax.nn.relu(x @ W_pre + b_pre)
    agg1 = jax.ops.segment_sum(h0[src], dst, num_segments=N)
    h1 = jax.nn.relu(agg1 @ W_nbr1 + h0 @ W_self1 + b1)
    x2 = jnp.concatenate([h1, h0], axis=1)
    agg_s = jax.ops.segment_sum(x2[src], dst, num_segments=N)
    score = (agg_s @ W_score_nbr + x2 @ W_score_self + b_score).squeeze(-1)

    xp_idx, nmap, xu1, xu0, p2a, p2b, s2 = _tc3(score, h1, h0, W_nbr2,
                                                W_self2, b2)

    pad_src = jnp.zeros((EP - E,), I32)
    pad_dst = DUMMY + (jnp.arange(EP - E, dtype=I32) % (NACC - N))
    srcflat = jnp.concatenate([src, pad_src])
    dstflat = jnp.concatenate([dst, pad_dst])
    src2d = srcflat.reshape(NCHUNK, 128)
    dst2d = dstflat.reshape(NCHUNK, 128)
    zeros = jnp.zeros((640, 128), F32)
    ivals = jnp.arange(N, dtype=I32)
    nm_pad = jnp.concatenate([nmap.reshape(N), jnp.full((NACC - N,), -1, I32)])

    s_ext, xpa, xpb = _sc_pool(xp_idx.reshape(N), ivals, xu1, xu0)
    agga, aggb = _sc_segsum(p2a, p2b, src2d, dst2d, zeros)
    a_ext = _sc_apool(nm_pad, src2d, dst2d)

    out = _tc4(agga, aggb, s2, xu1, xu0, W_post, b_post)

    x_pool = jnp.concatenate([xpa[:K], xpb[:K]], axis=1)
    a_pool = a_ext[:, :E]
    s_perm = s_ext[:K]
    return (out, edge_index, x_pool, a_pool, s_perm)


# speedup vs baseline: 1.4850x; 1.0037x over previous
import functools

import jax
import jax.numpy as jnp
from jax import lax
from jax.experimental import pallas as pl
from jax.experimental.pallas import tpu as pltpu
from jax.experimental.pallas import tpu_sc as plsc

N = 10000
E = 320000
K = 5000
H = 256
EP = 327680
NCHUNK = 2560
NACC = 10240
DUMMY = N
F32 = jnp.float32
I32 = jnp.int32


def _tc1_body(x_ref, wpre_ref, bpre_ref, h0_ref):
    h0_ref[...] = jax.nn.relu(
        jnp.dot(x_ref[...], wpre_ref[...], preferred_element_type=F32)
        + bpre_ref[...])


def _tc1(x, W_pre, b_pre):
    return pl.pallas_call(
        _tc1_body,
        grid=(10,),
        in_specs=[
            pl.BlockSpec((1000, 128), lambda i: (i, 0)),
            pl.BlockSpec((128, H), lambda i: (0, 0)),
            pl.BlockSpec((1, H), lambda i: (0, 0)),
        ],
        out_specs=pl.BlockSpec((1000, H), lambda i: (i, 0)),
        out_shape=jax.ShapeDtypeStruct((N, H), F32),
    )(x, W_pre, b_pre.reshape(1, H))


def _tc2_body(agg_ref, h0_ref, wn_ref, ws_ref, b_ref, h1_ref):
    h1_ref[...] = jax.nn.relu(
        jnp.dot(agg_ref[...], wn_ref[...], preferred_element_type=F32)
        + jnp.dot(h0_ref[...], ws_ref[...], preferred_element_type=F32)
        + b_ref[...])


def _tc2(agg1, h0, W_nbr1, W_self1, b1):
    return pl.pallas_call(
        _tc2_body,
        grid=(10,),
        in_specs=[
            pl.BlockSpec((1000, H), lambda i: (i, 0)),
            pl.BlockSpec((1000, H), lambda i: (i, 0)),
            pl.BlockSpec((H, H), lambda i: (0, 0)),
            pl.BlockSpec((H, H), lambda i: (0, 0)),
            pl.BlockSpec((1, H), lambda i: (0, 0)),
        ],
        out_specs=pl.BlockSpec((1000, H), lambda i: (i, 0)),
        out_shape=jax.ShapeDtypeStruct((N, H), F32),
    )(agg1, h0, W_nbr1, W_self1, b1.reshape(1, H))


def _tcscore_body(aggs_ref, x2_ref, wsn_ref, wss_ref, bs_ref, score_ref):
    score_ref[...] = (
        jnp.dot(aggs_ref[...], wsn_ref[...], preferred_element_type=F32)
        + jnp.dot(x2_ref[...], wss_ref[...], preferred_element_type=F32)
        + bs_ref[...])


def _tcscore(agg_s, x2, W_score_nbr, W_score_self, b_score):
    return pl.pallas_call(
        _tcscore_body,
        grid=(10,),
        in_specs=[
            pl.BlockSpec((1000, 2 * H), lambda i: (i, 0)),
            pl.BlockSpec((1000, 2 * H), lambda i: (i, 0)),
            pl.BlockSpec((2 * H, 1), lambda i: (0, 0)),
            pl.BlockSpec((2 * H, 1), lambda i: (0, 0)),
            pl.BlockSpec((1, 1), lambda i: (0, 0)),
        ],
        out_specs=pl.BlockSpec((1000, 1), lambda i: (i, 0)),
        out_shape=jax.ShapeDtypeStruct((N, 1), F32),
    )(agg_s, x2, W_score_nbr, W_score_self, b_score.reshape(1, 1))


def _tc3_body(score_col_ref, score_row_ref, h1_ref, h0_ref, wn2_ref, ws2_ref,
              b2_ref, xp_idx_ref, nmap_ref, xu1_ref, xu0_ref, p2a_ref, p2b_ref,
              s2_ref):
    i = pl.program_id(0)
    si = score_col_ref[...]
    iglob = lax.broadcasted_iota(I32, (1000, 1), 0) + i * 1000

    def jstep(c, rank):
        sj = score_row_ref[:, pl.ds(c * 1280, 1280)]
        jglob = lax.broadcasted_iota(I32, (1, 1280), 1) + c * 1280
        m = (sj > si) | ((sj == si) & (jglob < iglob))
        return rank + jnp.sum(m.astype(I32), axis=1, keepdims=True)

    rank = lax.fori_loop(0, 8, jstep, jnp.zeros((1000, 1), I32))
    valid = rank < K
    g = jnp.where(valid, jnp.tanh(si), 0.0)
    xp_idx_ref[...] = jnp.where(valid, rank, K + (rank & 7))
    nmap_ref[...] = jnp.where(valid, rank, -1)
    xu1 = h1_ref[...] * g
    xu0 = h0_ref[...] * g
    xu1_ref[...] = xu1
    xu0_ref[...] = xu0
    p2 = (jnp.dot(xu1, wn2_ref[:H], preferred_element_type=F32)
          + jnp.dot(xu0, wn2_ref[H:], preferred_element_type=F32))
    p2a_ref[...] = p2[:, :128]
    p2b_ref[...] = p2[:, 128:]
    s2_ref[...] = (jnp.dot(xu1, ws2_ref[:H], preferred_element_type=F32)
                   + jnp.dot(xu0, ws2_ref[H:], preferred_element_type=F32)
                   + b2_ref[...])


def _tc3(score, h1, h0, W_nbr2, W_self2, b2):
    score_col = score.reshape(N, 1)
    score_row = jnp.concatenate(
        [score, jnp.full((240,), -1e30, F32)]).reshape(1, N + 240)
    return pl.pallas_call(
        _tc3_body,
        grid=(10,),
        in_specs=[
            pl.BlockSpec((1000, 1), lambda i: (i, 0)),
            pl.BlockSpec((1, N + 240), lambda i: (0, 0)),
            pl.BlockSpec((1000, H), lambda i: (i, 0)),
            pl.BlockSpec((1000, H), lambda i: (i, 0)),
            pl.BlockSpec((2 * H, H), lambda i: (0, 0)),
            pl.BlockSpec((2 * H, H), lambda i: (0, 0)),
            pl.BlockSpec((1, H), lambda i: (0, 0)),
        ],
        out_specs=[
            pl.BlockSpec((1000, 1), lambda i: (i, 0)),
            pl.BlockSpec((1000, 1), lambda i: (i, 0)),
            pl.BlockSpec((1000, H), lambda i: (i, 0)),
            pl.BlockSpec((1000, H), lambda i: (i, 0)),
            pl.BlockSpec((1000, 128), lambda i: (i, 0)),
            pl.BlockSpec((1000, 128), lambda i: (i, 0)),
            pl.BlockSpec((1000, H), lambda i: (i, 0)),
        ],
        out_shape=[
            jax.ShapeDtypeStruct((N, 1), I32),
            jax.ShapeDtypeStruct((N, 1), I32),
            jax.ShapeDtypeStruct((N, H), F32),
            jax.ShapeDtypeStruct((N, H), F32),
            jax.ShapeDtypeStruct((N, 128), F32),
            jax.ShapeDtypeStruct((N, 128), F32),
            jax.ShapeDtypeStruct((N, H), F32),
        ],
    )(score_col, score_row, h1, h0, W_nbr2, W_self2, b2.reshape(1, H))


_MESH = plsc.VectorSubcoreMesh(core_axis_name="c", subcore_axis_name="s",
                               num_cores=2, num_subcores=16)


def _segsum_work(tab, src2d, dst2d, zeros, out, acc, srcb, dstb, rowb, s):
    pltpu.sync_copy(zeros, acc.at[pl.ds(s * 640, 640)])
    plsc.subcore_barrier()

    def group(gi, carry):
        base = s * 160 + gi * 16
        pltpu.sync_copy(src2d.at[pl.ds(base, 16)], srcb)
        pltpu.sync_copy(dst2d.at[pl.ds(base, 16)], dstb)

        def body(j, c2):
            pltpu.sync_copy(tab.at[srcb.at[j]], rowb)
            pltpu.sync_copy(rowb, acc.at[dstb.at[j]], add=True)
            return c2

        lax.fori_loop(0, 16, body, 0)
        return carry

    lax.fori_loop(0, 10, group, 0)
    plsc.subcore_barrier()
    pltpu.sync_copy(acc.at[pl.ds(s * 640, 640)], out.at[pl.ds(s * 640, 640)])


@functools.partial(
    pl.kernel,
    out_type=(jax.ShapeDtypeStruct((NACC, 128), F32),
              jax.ShapeDtypeStruct((NACC, 128), F32)),
    mesh=_MESH,
    scratch_types=[
        pltpu.VMEM_SHARED((NACC, 128), F32),
        pltpu.VMEM((16, 128), I32),
        pltpu.VMEM((16, 128), I32),
        pltpu.VMEM((128, 128), F32),
    ],
)
def _sc_segsum(pa, pb, src2d, dst2d, zeros, out_a, out_b, acc, srcb, dstb,
               rowb):
    c = lax.axis_index("c")
    s = lax.axis_index("s")

    @pl.when(c == 0)
    def _():
        _segsum_work(pa, src2d, dst2d, zeros, out_a, acc, srcb, dstb, rowb, s)

    @pl.when(c == 1)
    def _():
        _segsum_work(pb, src2d, dst2d, zeros, out_b, acc, srcb, dstb, rowb, s)


@functools.partial(
    pl.kernel,
    out_type=(jax.ShapeDtypeStruct((K + 8,), I32),
              jax.ShapeDtypeStruct((K + 8, H), F32),
              jax.ShapeDtypeStruct((K + 8, H), F32)),
    mesh=_MESH,
    scratch_types=[
        pltpu.VMEM((4, 80), I32),
        pltpu.VMEM((80,), I32),
        pltpu.VMEM((80, H), F32),
    ],
)
def _sc_pool(xp_idx, ivals, xu1, xu0, s_out, xpa, xpb, idxb, ivb, rowb):
    c = lax.axis_index("c")
    s = lax.axis_index("s")
    wid = s * 2 + c

    def chunk(j, carry):
        cid = wid + 32 * j

        @pl.when(cid < 125)
        def _():
            base = cid * 80
            pltpu.sync_copy(xp_idx.at[pl.ds(base, 80)], idxb.at[j])
            pltpu.sync_copy(ivals.at[pl.ds(base, 80)], ivb)
            pltpu.sync_copy(ivb, s_out.at[idxb.at[j]])
            pltpu.sync_copy(xu1.at[pl.ds(base, 80)], rowb)
            pltpu.sync_copy(rowb, xpa.at[idxb.at[j]])
            pltpu.sync_copy(xu0.at[pl.ds(base, 80)], rowb)
            pltpu.sync_copy(rowb, xpb.at[idxb.at[j]])

        return carry

    lax.fori_loop(0, 4, chunk, 0)


@functools.partial(
    pl.kernel,
    out_type=jax.ShapeDtypeStruct((2, EP), I32),
    mesh=_MESH,
    scratch_types=[
        pltpu.VMEM_SHARED((NACC,), I32),
        pltpu.VMEM((16, 128), I32),
        pltpu.VMEM((16, 128), I32),
        pltpu.VMEM((128,), I32),
        pltpu.VMEM((128,), I32),
        pltpu.VMEM((10240,), I32),
        pltpu.VMEM((10240,), I32),
    ],
)
def _sc_apool(nm_pad, src2d, dst2d, a_out, nmtab, sbuf, dbuf, smb, dmb, osb,
              odb):
    c = lax.axis_index("c")
    s = lax.axis_index("s")
    wid = s * 2 + c

    @pl.when(s == 0)
    def _():
        pltpu.sync_copy(nm_pad, nmtab)

    plsc.subcore_barrier()
    base_cid = wid * 80

    def group(g, carry):
        pltpu.sync_copy(src2d.at[pl.ds(base_cid + g * 16, 16)], sbuf)
        pltpu.sync_copy(dst2d.at[pl.ds(base_cid + g * 16, 16)], dbuf)

        def chunk(j, c2):
            pltpu.sync_copy(nmtab.at[sbuf.at[j]], smb)
            pltpu.sync_copy(nmtab.at[dbuf.at[j]], dmb)

            def vec(t, c3):
                o = t * 16
                sm = smb[pl.ds(o, 16)]
                dm = dmb[pl.ds(o, 16)]
                ok = (sm >= 0) & (dm >= 0)
                off = (g * 16 + j) * 128 + o
                osb[pl.ds(off, 16)] = jnp.where(ok, sm, -1)
                odb[pl.ds(off, 16)] = jnp.where(ok, dm, -1)
                return c3

            lax.fori_loop(0, 8, vec, 0)
            return c2

        lax.fori_loop(0, 16, chunk, 0)
        return carry

    lax.fori_loop(0, 5, group, 0)
    pltpu.sync_copy(osb, a_out.at[0, pl.ds(wid * 10240, 10240)])
    pltpu.sync_copy(odb, a_out.at[1, pl.ds(wid * 10240, 10240)])


def _tc4_body(agga_ref, aggb_ref, s2_ref, xu1_ref, xu0_ref, wp_ref, bp_ref,
              out_ref):
    agg = jnp.concatenate([agga_ref[...], aggb_ref[...]], axis=1)
    h2 = jax.nn.relu(agg + s2_ref[...])
    out_ref[...] = (jnp.dot(h2, wp_ref[:H], preferred_element_type=F32)
                    + jnp.dot(xu1_ref[...], wp_ref[H:2 * H],
                              preferred_element_type=F32)
                    + jnp.dot(xu0_ref[...], wp_ref[2 * H:],
                              preferred_element_type=F32)
                    + bp_ref[...])


def _tc4(agga, aggb, s2, xu1, xu0, W_post, b_post):
    return pl.pallas_call(
        _tc4_body,
        grid=(10,),
        in_specs=[
            pl.BlockSpec((1000, 128), lambda i: (i, 0)),
            pl.BlockSpec((1000, 128), lambda i: (i, 0)),
            pl.BlockSpec((1000, H), lambda i: (i, 0)),
            pl.BlockSpec((1000, H), lambda i: (i, 0)),
            pl.BlockSpec((1000, H), lambda i: (i, 0)),
            pl.BlockSpec((3 * H, 128), lambda i: (0, 0)),
            pl.BlockSpec((1, 128), lambda i: (0, 0)),
        ],
        out_specs=pl.BlockSpec((1000, 128), lambda i: (i, 0)),
        out_shape=jax.ShapeDtypeStruct((N, 128), F32),
    )(agga, aggb, s2, xu1, xu0, W_post, b_post.reshape(1, 128))


def kernel(x, edge_index, batch, epoch, W_pre, b_pre, W_nbr1, W_self1, b1,
           W_score_nbr, W_score_self, b_score, W_nbr2, W_self2, b2, W_post,
           b_post):
    src, dst = edge_index[0], edge_index[1]

    h0 = _tc1(x, W_pre, b_pre)
    agg1 = jax.ops.segment_sum(h0[src], dst, num_segments=N)
    h1 = _tc2(agg1, h0, W_nbr1, W_self1, b1)
    x2 = jnp.concatenate([h1, h0], axis=1)
    agg_s = jax.ops.segment_sum(x2[src], dst, num_segments=N)
    score = _tcscore(agg_s, x2, W_score_nbr, W_score_self,
                     b_score).reshape(N)

    xp_idx, nmap, xu1, xu0, p2a, p2b, s2 = _tc3(score, h1, h0, W_nbr2,
                                                W_self2, b2)

    pad_src = jnp.zeros((EP - E,), I32)
    pad_dst = DUMMY + (jnp.arange(EP - E, dtype=I32) % (NACC - N))
    srcflat = jnp.concatenate([src, pad_src])
    dstflat = jnp.concatenate([dst, pad_dst])
    src2d = srcflat.reshape(NCHUNK, 128)
    dst2d = dstflat.reshape(NCHUNK, 128)
    zeros = jnp.zeros((640, 128), F32)
    ivals = jnp.arange(N, dtype=I32)
    nm_pad = jnp.concatenate([nmap.reshape(N), jnp.full((NACC - N,), -1, I32)])

    s_ext, xpa, xpb = _sc_pool(xp_idx.reshape(N), ivals, xu1, xu0)
    agga, aggb = _sc_segsum(p2a, p2b, src2d, dst2d, zeros)
    a_ext = _sc_apool(nm_pad, src2d, dst2d)

    out = _tc4(agga, aggb, s2, xu1, xu0, W_post, b_post)

    x_pool = jnp.concatenate([xpa[:K], xpb[:K]], axis=1)
    a_pool = a_ext[:, :E]
    s_perm = s_ext[:K]
    return (out, edge_index, x_pool, a_pool, s_perm)
